# Initial kernel scaffold; baseline (speedup 1.0000x reference)
import jax
import jax.numpy as jnp
from jax.experimental import pallas as pl


def kernel(x_q, x_edge, edge_ids, w_q):
    raise NotImplementedError("write your pallas kernel here")


# speedup vs baseline: 1.0938x; 1.0938x over previous
import jax
import jax.numpy as jnp
from jax import lax
from jax.experimental import pallas as pl
from jax.experimental.pallas import tpu as pltpu

E_BLK = 256
C = 512
D = 256


def _body(starts_ref, ids_ref, xq_ref, x_edge_ref, w_ref, out_ref,
          xbuf0, xbuf1, idbuf0, idbuf1, agg_ref, sems):
    i = pl.program_id(0)
    nq = ids_ref.shape[0]
    s = starts_ref[i]
    e = starts_ref[i + 1]
    s0 = (s // 8) * 8
    ntrips = (e - s0 + C - 1) // C

    agg_ref[...] = jnp.zeros_like(agg_ref)

    xbufs = (xbuf0, xbuf1)
    idbufs = (idbuf0, idbuf1)

    def chunk_base(c):
        return jnp.minimum(s0 + c * C, nq - C)

    def start_copy(c, slot):
        base = chunk_base(c)
        pltpu.make_async_copy(
            xq_ref.at[pl.ds(base, C), :], xbufs[slot], sems.at[slot, 0]).start()
        pltpu.make_async_copy(
            ids_ref.at[pl.ds(base, C), :], idbufs[slot], sems.at[slot, 1]).start()

    def do_chunk(c, slot):
        base = chunk_base(c)
        pltpu.make_async_copy(
            xq_ref.at[pl.ds(base, C), :], xbufs[slot], sems.at[slot, 0]).wait()
        pltpu.make_async_copy(
            ids_ref.at[pl.ds(base, C), :], idbufs[slot], sems.at[slot, 1]).wait()
        idc = idbufs[slot][...]
        xc = xbufs[slot][...]
        rel = idc - i * E_BLK
        lane = lax.broadcasted_iota(jnp.int32, (C, E_BLK), 1)
        row = base + lax.broadcasted_iota(jnp.int32, (C, E_BLK), 0)
        onehot = ((rel == lane) & (row >= s0 + c * C)).astype(jnp.float32)
        agg_ref[...] += lax.dot_general(
            onehot, xc, (((0,), (0,)), ((), ())),
            preferred_element_type=jnp.float32)

    @pl.when(ntrips > 0)
    def _():
        start_copy(0, 0)

    def loop_body(c, carry):
        even = lax.rem(c, 2) == 0

        @pl.when(c + 1 < ntrips)
        def _():
            @pl.when(even)
            def _():
                start_copy(c + 1, 1)

            @pl.when(jnp.logical_not(even))
            def _():
                start_copy(c + 1, 0)

        @pl.when(even)
        def _():
            do_chunk(c, 0)

        @pl.when(jnp.logical_not(even))
        def _():
            do_chunk(c, 1)

        return carry

    lax.fori_loop(0, ntrips, loop_body, 0)

    out_ref[...] = (
        jnp.dot(x_edge_ref[...], w_ref[0:D, :], preferred_element_type=jnp.float32)
        + jnp.dot(agg_ref[...], w_ref[D:2 * D, :], preferred_element_type=jnp.float32))


def kernel(x_q, x_edge, edge_ids, w_q):
    num_edges = x_edge.shape[0]
    ids32 = edge_ids.astype(jnp.int32)
    bounds = jnp.arange(0, num_edges + 1, E_BLK, dtype=jnp.int32)
    starts = jnp.searchsorted(ids32, bounds).astype(jnp.int32)
    ids_col = ids32.reshape(-1, 1)
    grid = num_edges // E_BLK

    return pl.pallas_call(
        _body,
        grid=(grid,),
        in_specs=[
            pl.BlockSpec(memory_space=pltpu.SMEM),
            pl.BlockSpec(memory_space=pl.ANY),
            pl.BlockSpec(memory_space=pl.ANY),
            pl.BlockSpec((E_BLK, D), lambda i: (i, 0)),
            pl.BlockSpec((2 * D, D), lambda i: (0, 0)),
        ],
        out_specs=pl.BlockSpec((E_BLK, D), lambda i: (i, 0)),
        out_shape=jax.ShapeDtypeStruct((num_edges, D), jnp.float32),
        scratch_shapes=[
            pltpu.VMEM((C, D), jnp.float32),
            pltpu.VMEM((C, D), jnp.float32),
            pltpu.VMEM((C, 1), jnp.int32),
            pltpu.VMEM((C, 1), jnp.int32),
            pltpu.VMEM((E_BLK, D), jnp.float32),
            pltpu.SemaphoreType.DMA((2, 2)),
        ],
        compiler_params=pltpu.CompilerParams(
            dimension_semantics=("arbitrary",)),
    )(starts, ids_col, x_q, x_edge, w_q)
